# baseline (speedup 1.0000x reference)
import functools

import jax
import jax.numpy as jnp
from jax import lax
from jax.experimental import pallas as pl
from jax.experimental.pallas import tpu as pltpu
from jax.experimental.pallas import tpu_sc as plsc

EMBED_DIM = 64
NUM_NEG = 200
N_TARGETS = 1 + NUM_NEG
PER_TILE = 7


def _softplus(z):
    e = jnp.exp(-jnp.abs(z))
    t = e / (e + 2.0)
    t2 = t * t
    ln_y = 2.0 * t * (1.0 + t2 * (1.0 / 3.0 + t2 * (1.0 / 5.0 + t2 * (1.0 / 7.0 + t2 * (1.0 / 9.0)))))
    return jnp.maximum(z, 0.0) + ln_y


def _lane_shuffle(v, idxs):
    return lax.gather(
        v, idxs[:, None],
        dimension_numbers=lax.GatherDimensionNumbers(
            offset_dims=(), collapsed_slice_dims=(0,), start_index_map=(0,)),
        slice_sizes=(1,),
        mode=lax.GatherScatterMode.PROMISE_IN_BOUNDS)


def _sc_kernel(inp_t_hbm, out_t_hbm, cen_hbm, pos_hbm, neg_hbm, out_hbm,
               idx_v, cen_v, ce_buf, win_buf, sp_v, sp_shared, red_buf,
               out_v, sem):
    core = lax.axis_index("c")
    s = lax.axis_index("s")
    wid = s * 2 + core
    lanes = lax.iota(jnp.int32, 16)

    zeros16 = jnp.zeros((16,), jnp.int32)
    idx_v[pl.ds(192, 16)] = zeros16
    idx_v[pl.ds(208, 16)] = zeros16
    idx_v[pl.ds(224, 16)] = zeros16
    idx_v[pl.ds(240, 16)] = zeros16
    pltpu.sync_copy(neg_hbm, idx_v.at[pl.ds(0, NUM_NEG)])
    pltpu.sync_copy(pos_hbm, idx_v.at[pl.ds(NUM_NEG, 1)])
    pltpu.sync_copy(cen_hbm, cen_v.at[pl.ds(0, 1)])

    c = cen_v[...][0]
    cp_ce = pltpu.make_async_copy(
        inp_t_hbm.at[:, pl.ds(pl.multiple_of((c >> 7) << 7, 128), 128)],
        ce_buf, sem)
    cp_ce.start()

    base = wid * PER_TILE
    sh = jnp.bitwise_and(base, 15)
    v0 = idx_v[pl.ds(jnp.bitwise_and(base, ~15), 16)]
    v1 = idx_v[pl.ds(jnp.bitwise_and(base, ~15) + 16, 16)]
    rot = jnp.bitwise_and(lanes + sh, 15)
    g0 = _lane_shuffle(v0, rot)
    g1 = _lane_shuffle(v1, rot)
    idx_vec = jnp.where(lanes + sh < 16, g0, g1)

    cps = []
    for i in range(PER_TILE):
        r = idx_vec[i]
        cp = pltpu.make_async_copy(
            out_t_hbm.at[:, pl.ds(pl.multiple_of((r >> 7) << 7, 128), 128)],
            win_buf.at[pl.ds(i * EMBED_DIM, EMBED_DIM)], sem)
        cp.start()
        cps.append(cp)
    cp_ce.wait()
    for cp in cps:
        cp.wait()

    off_vec = jnp.bitwise_and(idx_vec, 127)
    i_vec = jnp.where(lanes < PER_TILE, lanes, 0)
    row_base = i_vec * EMBED_DIM
    c_off = jnp.full((16,), jnp.bitwise_and(c, 127), jnp.int32)

    def body(j, acc):
        jv = jnp.full((16,), j, jnp.int32)
        cej = plsc.load_gather(ce_buf, [jv, c_off])
        col = plsc.load_gather(win_buf, [row_base + jv, off_vec])
        return acc + cej * col

    acc = lax.fori_loop(0, EMBED_DIM, body, jnp.zeros((16,), jnp.float32),
                        unroll=8)

    t_vec = lanes + base
    d = jnp.where(t_vec == NUM_NEG, -acc, acc)
    valid = jnp.logical_and(lanes < PER_TILE, t_vec < N_TARGETS)
    d = jnp.where(valid, d, -1e30)
    sp_v[...] = _softplus(d)
    pltpu.sync_copy(sp_v, sp_shared.at[pl.ds(s * 16, 16)])

    plsc.subcore_barrier()

    @pl.when(s == 0)
    def _():
        pltpu.sync_copy(sp_shared, red_buf)
        total = red_buf[pl.ds(0, 16)]
        for ww in range(1, 16):
            total = total + red_buf[pl.ds(ww * 16, 16)]
        out_v[...] = jnp.full((16,), jnp.sum(total))
        pltpu.sync_copy(out_v.at[pl.ds(0, 8)],
                        out_hbm.at[pl.ds(pl.multiple_of(core * 8, 8), 8)])


@jax.jit
def _run(center_word, positive_words, negative_words, input_embeddings, output_embeddings):
    inp_t = input_embeddings.T
    out_t = output_embeddings.T
    mesh = plsc.VectorSubcoreMesh(core_axis_name="c", subcore_axis_name="s")
    k = functools.partial(
        pl.kernel,
        mesh=mesh,
        compiler_params=pltpu.CompilerParams(needs_layout_passes=False),
        out_type=jax.ShapeDtypeStruct((16,), jnp.float32),
        scratch_types=[
            pltpu.VMEM((256,), jnp.int32),
            pltpu.VMEM((16,), jnp.int32),
            pltpu.VMEM((EMBED_DIM, 128), jnp.float32),
            pltpu.VMEM((PER_TILE * EMBED_DIM, 128), jnp.float32),
            pltpu.VMEM((16,), jnp.float32),
            pltpu.VMEM_SHARED((256,), jnp.float32),
            pltpu.VMEM((256,), jnp.float32),
            pltpu.VMEM((16,), jnp.float32),
            pltpu.SemaphoreType.DMA,
        ],
    )(_sc_kernel)
    res = k(inp_t, out_t, center_word.astype(jnp.int32),
            positive_words.astype(jnp.int32), negative_words.astype(jnp.int32))
    return (res[0] + res[8]).reshape(1, 1)


def kernel(center_word, positive_words, negative_words, input_embeddings, output_embeddings):
    return _run(center_word, positive_words, negative_words,
                input_embeddings, output_embeddings)


# speedup vs baseline: 1.1182x; 1.1182x over previous
import functools

import jax
import jax.numpy as jnp
from jax import lax
from jax.experimental import pallas as pl
from jax.experimental.pallas import tpu as pltpu
from jax.experimental.pallas import tpu_sc as plsc

EMBED_DIM = 64
NUM_NEG = 200
N_TARGETS = 1 + NUM_NEG
PER_TILE = 13


def _softplus(z):
    e = jnp.exp(-jnp.abs(z))
    t = e / (e + 2.0)
    t2 = t * t
    ln_y = 2.0 * t * (1.0 + t2 * (1.0 / 3.0 + t2 * (1.0 / 5.0 + t2 * (1.0 / 7.0 + t2 * (1.0 / 9.0)))))
    return jnp.maximum(z, 0.0) + ln_y


def _lane_shuffle(v, idxs):
    return lax.gather(
        v, idxs[:, None],
        dimension_numbers=lax.GatherDimensionNumbers(
            offset_dims=(), collapsed_slice_dims=(0,), start_index_map=(0,)),
        slice_sizes=(1,),
        mode=lax.GatherScatterMode.PROMISE_IN_BOUNDS)


def _sc_kernel(inp_t_hbm, out_t_hbm, cen_hbm, pos_hbm, neg_hbm, out_hbm,
               idx_v, cen_v, ce_buf, win_buf, sp_v, sp_shared, red_buf,
               out_v, sem):
    on_core0 = lax.axis_index("c") == 0
    lanes = lax.iota(jnp.int32, 16)

    @pl.when(on_core0)
    def _():
        w = lax.axis_index("s")

        idx_v[pl.ds(192, 16)] = jnp.zeros((16,), jnp.int32)
        pltpu.sync_copy(neg_hbm, idx_v.at[pl.ds(0, NUM_NEG)])
        pltpu.sync_copy(pos_hbm, idx_v.at[pl.ds(NUM_NEG, 1)])
        pltpu.sync_copy(cen_hbm, cen_v.at[pl.ds(0, 1)])

        c = cen_v[...][0]
        cp_ce = pltpu.make_async_copy(
            inp_t_hbm.at[:, pl.ds(pl.multiple_of((c >> 7) << 7, 128), 128)],
            ce_buf, sem)
        cp_ce.start()

        base = w * PER_TILE
        sh = jnp.bitwise_and(base, 15)
        v0 = idx_v[pl.ds(jnp.bitwise_and(base, ~15), 16)]
        v1 = idx_v[pl.ds(jnp.bitwise_and(base, ~15) + 16, 16)]
        rot = jnp.bitwise_and(lanes + sh, 15)
        g0 = _lane_shuffle(v0, rot)
        g1 = _lane_shuffle(v1, rot)
        idx_vec = jnp.where(lanes + sh < 16, g0, g1)

        cps = []
        for i in range(PER_TILE):
            r = idx_vec[i]
            cp = pltpu.make_async_copy(
                out_t_hbm.at[:, pl.ds(pl.multiple_of((r >> 7) << 7, 128), 128)],
                win_buf.at[pl.ds(i * EMBED_DIM, EMBED_DIM)], sem)
            cp.start()
            cps.append(cp)
        cp_ce.wait()
        for cp in cps:
            cp.wait()

        off_vec = jnp.bitwise_and(idx_vec, 127)
        i_vec = jnp.where(lanes < PER_TILE, lanes, 0)
        row_base = i_vec * EMBED_DIM
        c_off = jnp.full((16,), jnp.bitwise_and(c, 127), jnp.int32)

        def body(j, acc):
            jv = jnp.full((16,), j, jnp.int32)
            cej = plsc.load_gather(ce_buf, [jv, c_off])
            col = plsc.load_gather(win_buf, [row_base + jv, off_vec])
            return acc + cej * col

        acc = lax.fori_loop(0, EMBED_DIM, body, jnp.zeros((16,), jnp.float32),
                            unroll=8)

        t_vec = lanes + w * PER_TILE
        d = jnp.where(t_vec == NUM_NEG, -acc, acc)
        valid = jnp.logical_and(lanes < PER_TILE, t_vec < N_TARGETS)
        d = jnp.where(valid, d, -1e30)
        sp_v[...] = _softplus(d)
        pltpu.sync_copy(sp_v, sp_shared.at[pl.ds(w * 16, 16)])

    plsc.subcore_barrier()

    @pl.when(jnp.logical_and(on_core0, lax.axis_index("s") == 0))
    def _():
        pltpu.sync_copy(sp_shared, red_buf)
        total = red_buf[pl.ds(0, 16)]
        for ww in range(1, 16):
            total = total + red_buf[pl.ds(ww * 16, 16)]
        out_v[...] = jnp.full((16,), jnp.sum(total))
        pltpu.sync_copy(out_v.at[pl.ds(0, 1)], out_hbm.at[0])


@jax.jit
def _run(center_word, positive_words, negative_words, input_embeddings, output_embeddings):
    inp_t = input_embeddings.T
    out_t = output_embeddings.T
    mesh = plsc.VectorSubcoreMesh(core_axis_name="c", subcore_axis_name="s")
    k = functools.partial(
        pl.kernel,
        mesh=mesh,
        compiler_params=pltpu.CompilerParams(needs_layout_passes=False),
        out_type=jax.ShapeDtypeStruct((1, 1), jnp.float32),
        scratch_types=[
            pltpu.VMEM((208,), jnp.int32),
            pltpu.VMEM((16,), jnp.int32),
            pltpu.VMEM((EMBED_DIM, 128), jnp.float32),
            pltpu.VMEM((PER_TILE * EMBED_DIM, 128), jnp.float32),
            pltpu.VMEM((16,), jnp.float32),
            pltpu.VMEM_SHARED((256,), jnp.float32),
            pltpu.VMEM((256,), jnp.float32),
            pltpu.VMEM((16,), jnp.float32),
            pltpu.SemaphoreType.DMA,
        ],
    )(_sc_kernel)
    return k(inp_t, out_t, center_word.astype(jnp.int32),
             positive_words.astype(jnp.int32), negative_words.astype(jnp.int32))


def kernel(center_word, positive_words, negative_words, input_embeddings, output_embeddings):
    return _run(center_word, positive_words, negative_words,
                input_embeddings, output_embeddings)


# speedup vs baseline: 1.2129x; 1.0848x over previous
import functools

import jax
import jax.numpy as jnp
from jax import lax
from jax.experimental import pallas as pl
from jax.experimental.pallas import tpu as pltpu
from jax.experimental.pallas import tpu_sc as plsc

EMBED_DIM = 64
NUM_NEG = 200
N_TARGETS = 1 + NUM_NEG
PER_TILE = 13


def _softplus(z):
    e = jnp.exp(-jnp.abs(z))
    t = e / (e + 2.0)
    t2 = t * t
    ln_y = 2.0 * t * (1.0 + t2 * (1.0 / 3.0 + t2 * (1.0 / 5.0 + t2 * (1.0 / 7.0 + t2 * (1.0 / 9.0)))))
    return jnp.maximum(z, 0.0) + ln_y


def _lane_shuffle(v, idxs):
    return lax.gather(
        v, idxs[:, None],
        dimension_numbers=lax.GatherDimensionNumbers(
            offset_dims=(), collapsed_slice_dims=(0,), start_index_map=(0,)),
        slice_sizes=(1,),
        mode=lax.GatherScatterMode.PROMISE_IN_BOUNDS)


def _sc_kernel(inp_t_hbm, out_t_hbm, cen_hbm, pos_hbm, neg_hbm, out_hbm,
               idx_v, cen_v, ce_buf, win_buf, sp_v, sp_shared, red_buf,
               out_v, sem, sem_i, sem_c):
    on_core0 = lax.axis_index("c") == 0
    lanes = lax.iota(jnp.int32, 16)

    @pl.when(on_core0)
    def _():
        w = lax.axis_index("s")

        idx_v[pl.ds(192, 16)] = jnp.zeros((16,), jnp.int32)
        st0 = pltpu.make_async_copy(neg_hbm, idx_v.at[pl.ds(0, NUM_NEG)], sem_i)
        st1 = pltpu.make_async_copy(pos_hbm, idx_v.at[pl.ds(NUM_NEG, 1)], sem_i)
        st2 = pltpu.make_async_copy(cen_hbm, cen_v.at[pl.ds(0, 1)], sem_c)
        st0.start()
        st1.start()
        st2.start()
        st2.wait()
        c = cen_v[...][0]
        cp_ce = pltpu.make_async_copy(
            inp_t_hbm.at[:, pl.ds(pl.multiple_of((c >> 7) << 7, 128), 128)],
            ce_buf, sem)
        cp_ce.start()

        st0.wait()
        st1.wait()
        base = w * PER_TILE
        sh = jnp.bitwise_and(base, 15)
        v0 = idx_v[pl.ds(jnp.bitwise_and(base, ~15), 16)]
        v1 = idx_v[pl.ds(jnp.bitwise_and(base, ~15) + 16, 16)]
        rot = jnp.bitwise_and(lanes + sh, 15)
        g0 = _lane_shuffle(v0, rot)
        g1 = _lane_shuffle(v1, rot)
        idx_vec = jnp.where(lanes + sh < 16, g0, g1)

        cps = []
        for i in range(PER_TILE):
            r = idx_vec[i]
            cp = pltpu.make_async_copy(
                out_t_hbm.at[:, pl.ds(pl.multiple_of((r >> 7) << 7, 128), 128)],
                win_buf.at[pl.ds(i * EMBED_DIM, EMBED_DIM)], sem)
            cp.start()
            cps.append(cp)
        cp_ce.wait()
        for cp in cps:
            cp.wait()

        off_vec = jnp.bitwise_and(idx_vec, 127)
        i_vec = jnp.where(lanes < PER_TILE, lanes, 0)
        row_base = i_vec * EMBED_DIM
        c_off = jnp.full((16,), jnp.bitwise_and(c, 127), jnp.int32)

        def body(j, acc):
            jv = jnp.full((16,), j, jnp.int32)
            cej = plsc.load_gather(ce_buf, [jv, c_off])
            col = plsc.load_gather(win_buf, [row_base + jv, off_vec])
            return acc + cej * col

        acc = lax.fori_loop(0, EMBED_DIM, body, jnp.zeros((16,), jnp.float32),
                            unroll=16)

        t_vec = lanes + w * PER_TILE
        d = jnp.where(t_vec == NUM_NEG, -acc, acc)
        valid = jnp.logical_and(lanes < PER_TILE, t_vec < N_TARGETS)
        d = jnp.where(valid, d, -1e30)
        sp_v[...] = _softplus(d)
        pltpu.sync_copy(sp_v, sp_shared.at[pl.ds(w * 16, 16)])

    plsc.subcore_barrier()

    @pl.when(jnp.logical_and(on_core0, lax.axis_index("s") == 0))
    def _():
        pltpu.sync_copy(sp_shared, red_buf)
        total = red_buf[pl.ds(0, 16)]
        for ww in range(1, 16):
            total = total + red_buf[pl.ds(ww * 16, 16)]
        out_v[...] = jnp.full((16,), jnp.sum(total))
        pltpu.sync_copy(out_v.at[pl.ds(0, 1)], out_hbm.at[0])


@jax.jit
def _run(center_word, positive_words, negative_words, input_embeddings, output_embeddings):
    inp_t = input_embeddings.T
    out_t = output_embeddings.T
    mesh = plsc.VectorSubcoreMesh(core_axis_name="c", subcore_axis_name="s",
                                  num_cores=1)
    k = functools.partial(
        pl.kernel,
        mesh=mesh,
        compiler_params=pltpu.CompilerParams(needs_layout_passes=False),
        out_type=jax.ShapeDtypeStruct((1, 1), jnp.float32),
        scratch_types=[
            pltpu.VMEM((208,), jnp.int32),
            pltpu.VMEM((16,), jnp.int32),
            pltpu.VMEM((EMBED_DIM, 128), jnp.float32),
            pltpu.VMEM((PER_TILE * EMBED_DIM, 128), jnp.float32),
            pltpu.VMEM((16,), jnp.float32),
            pltpu.VMEM_SHARED((256,), jnp.float32),
            pltpu.VMEM((256,), jnp.float32),
            pltpu.VMEM((16,), jnp.float32),
            pltpu.SemaphoreType.DMA,
            pltpu.SemaphoreType.DMA,
            pltpu.SemaphoreType.DMA,
        ],
    )(_sc_kernel)
    return k(inp_t, out_t, center_word.astype(jnp.int32),
             positive_words.astype(jnp.int32), negative_words.astype(jnp.int32))


def kernel(center_word, positive_words, negative_words, input_embeddings, output_embeddings):
    return _run(center_word, positive_words, negative_words,
                input_embeddings, output_embeddings)
